# baseline (speedup 1.0000x reference)
import functools

import jax
import jax.numpy as jnp
from jax.experimental import pallas as pl
from jax.experimental.pallas import tpu as pltpu

_EPS = 1e-5
_CC = 0.25
_LZ = 10.0
_DV1 = 1.0
_DV2 = 1.0


def _dot_t(a, b):
    return jax.lax.dot_general(
        a.astype(jnp.bfloat16), b.astype(jnp.bfloat16),
        (((1,), (1,)), ((), ())), preferred_element_type=jnp.float32)


def _bn_mish(h, g, beta):
    m = jnp.mean(h, axis=0, keepdims=True)
    v = jnp.mean((h - m) ** 2, axis=0, keepdims=True)
    h = (h - m) / jnp.sqrt(v + _EPS) * g + beta
    return h * jnp.tanh(jnp.logaddexp(h, 0.0))


def _layer_body(x_ref, w_ref, b_ref, g_ref, bt_ref, o_ref, acc_ref, *, nk, act):
    k = pl.program_id(0)

    @pl.when(k == 0)
    def _init():
        acc_ref[...] = jnp.zeros_like(acc_ref)

    acc_ref[...] += jax.lax.dot_general(
        x_ref[...], w_ref[...], (((1,), (1,)), ((), ())),
        preferred_element_type=jnp.float32)

    @pl.when(k == nk - 1)
    def _finish():
        h = acc_ref[...] + b_ref[...]
        if act:
            h = _bn_mish(h, g_ref[...], bt_ref[...])
        o_ref[...] = h.astype(o_ref.dtype)


def _layer(x, W, b, g, beta, act, k_blk=None, out_dtype=jnp.bfloat16):
    B, K = x.shape
    N = W.shape[0]
    if k_blk is None or k_blk > K:
        k_blk = K
    nk = K // k_blk
    if g is None:
        g = jnp.zeros((N,), jnp.float32)
        beta = jnp.zeros((N,), jnp.float32)
    body = functools.partial(_layer_body, nk=nk, act=act)
    return pl.pallas_call(
        body,
        grid=(nk,),
        in_specs=[
            pl.BlockSpec((B, k_blk), lambda k: (0, k)),
            pl.BlockSpec((N, k_blk), lambda k: (0, k)),
            pl.BlockSpec((1, N), lambda k: (0, 0)),
            pl.BlockSpec((1, N), lambda k: (0, 0)),
            pl.BlockSpec((1, N), lambda k: (0, 0)),
        ],
        out_specs=pl.BlockSpec((B, N), lambda k: (0, 0)),
        out_shape=jax.ShapeDtypeStruct((B, N), out_dtype),
        scratch_shapes=[pltpu.VMEM((B, N), jnp.float32)],
    )(x, W, b.reshape(1, N), g.reshape(1, N), beta.reshape(1, N))


def _final_body(x_ref, w_ref, b_ref, t_ref, o_ref):
    n = pl.program_id(0)
    h = jax.lax.dot_general(
        x_ref[...], w_ref[...], (((1,), (1,)), ((), ())),
        preferred_element_type=jnp.float32)
    d = (h + b_ref[...]) - t_ref[...]

    @pl.when(n == 0)
    def _init():
        o_ref[...] = jnp.zeros_like(o_ref)

    o_ref[...] += jnp.sum(d * d).reshape(1, 1)


def _final_layer_sse(x, W, b, target, n_blk=512):
    B, K = x.shape
    N = W.shape[0]
    nn = N // n_blk
    return pl.pallas_call(
        _final_body,
        grid=(nn,),
        in_specs=[
            pl.BlockSpec((B, K), lambda n: (0, 0)),
            pl.BlockSpec((n_blk, K), lambda n: (n, 0)),
            pl.BlockSpec((1, n_blk), lambda n: (0, n)),
            pl.BlockSpec((B, n_blk), lambda n: (0, n)),
        ],
        out_specs=pl.BlockSpec((1, 1), lambda n: (0, 0)),
        out_shape=jax.ShapeDtypeStruct((1, 1), jnp.float32),
    )(x, W, b.reshape(1, N), target)


def _vq_body(z_ref, z1_ref, cbx_ref, cby_ref, q_ref, q1_ref, s_ref):
    def one(z, cb):
        zz = jnp.sum(z * z, axis=1, keepdims=True)
        cc = jnp.sum(cb * cb, axis=1)[None, :]
        zc = _dot_t(z, cb)
        d = zz + cc - 2.0 * zc
        dmin = jnp.min(d, axis=1, keepdims=True)
        ids = jax.lax.broadcasted_iota(jnp.int32, d.shape, 1)
        cand = jnp.where(d <= dmin, ids, d.shape[1])
        idx = jnp.min(cand, axis=1, keepdims=True)
        oh = (ids == idx).astype(jnp.float32)
        q = jax.lax.dot_general(
            oh, cb, (((1,), (0,)), ((), ())), preferred_element_type=jnp.float32)
        sse = jnp.sum((q - z) ** 2)
        return q, sse

    z = z_ref[...]
    z1 = z1_ref[...]
    q, sse_x = one(z, cbx_ref[...])
    q1, sse_y = one(z1, cby_ref[...])
    q_ref[...] = q.astype(jnp.bfloat16)
    q1_ref[...] = q1.astype(jnp.bfloat16)
    denom = z.shape[0] * z.shape[1]
    s_ref[...] = (((1.0 + _CC) * (sse_x + sse_y)
                   + _LZ * jnp.sum((z - z1) ** 2)) / denom).reshape(1, 1)


def _vq_both(z, z1, cb_x, cb_y):
    B, E = z.shape
    return pl.pallas_call(
        _vq_body,
        out_shape=(
            jax.ShapeDtypeStruct((B, E), jnp.bfloat16),
            jax.ShapeDtypeStruct((B, E), jnp.bfloat16),
            jax.ShapeDtypeStruct((1, 1), jnp.float32),
        ),
    )(z, z1, cb_x, cb_y)


def _encoder(inp, p):
    h = _layer(inp, p["W"][0], p["b"][0], p["g"][0], p["beta"][0], True,
               k_blk=1024)
    h = _layer(h, p["W"][1], p["b"][1], p["g"][1], p["beta"][1], True)
    h = _layer(h, p["W"][2], p["b"][2], p["g"][2], p["beta"][2], True)
    h = _layer(h, p["W"][3], p["b"][3], None, None, False, out_dtype=jnp.float32)
    return h


def _decoder_sse(q, p, target):
    h = _layer(q, p["W"][0], p["b"][0], p["g"][0], p["beta"][0], True)
    h = _layer(h, p["W"][1], p["b"][1], p["g"][1], p["beta"][1], True)
    h = _layer(h, p["W"][2], p["b"][2], p["g"][2], p["beta"][2], True)
    return _final_layer_sse(h, p["W"][3], p["b"][3], target)


def _bf16_tree(p):
    return {
        "W": [w.astype(jnp.bfloat16) for w in p["W"]],
        "b": p["b"], "g": p["g"], "beta": p["beta"],
    }


def kernel(x, y, params):
    B, in_dim = x.shape
    px = _bf16_tree(params["enc_x"])
    py = _bf16_tree(params["enc_y"])
    pd = _bf16_tree(params["dec"])
    xb = x.astype(jnp.bfloat16)
    yb = y.astype(jnp.bfloat16)
    z = _encoder(xb, px)
    z1 = _encoder(yb, py)
    q, q1, s_vq = _vq_both(z, z1, params["cb_x"], params["cb_y"])
    sse_x = _decoder_sse(q, pd, y)
    sse_y = _decoder_sse(q1, pd, x)
    recon = (sse_x[0, 0] / _DV1 + sse_y[0, 0] / _DV2) / (B * in_dim)
    return s_vq[0, 0] + recon


# speedup vs baseline: 1.1461x; 1.1461x over previous
import functools

import jax
import jax.numpy as jnp
from jax.experimental import pallas as pl
from jax.experimental.pallas import tpu as pltpu

_EPS = 1e-5
_CC = 0.25
_LZ = 10.0
_DV1 = 1.0
_DV2 = 1.0


def _dot_nt(a, b):
    return jax.lax.dot_general(
        a, b, (((1,), (1,)), ((), ())), preferred_element_type=jnp.float32)


def _mish(h):
    s = jnp.exp(jnp.minimum(h, 30.0))
    p = s * (s + 2.0)
    return h * p / (p + 2.0)


def _bn(h, g, beta):
    m = jnp.mean(h, axis=0, keepdims=True)
    v = jnp.maximum(jnp.mean(h * h, axis=0, keepdims=True) - m * m, 0.0)
    a = g / jnp.sqrt(v + _EPS)
    return h * a + (beta - m * a)


def _layer_body(x_ref, w_ref, b_ref, g_ref, bt_ref, o_ref, *, act, halves):
    h = _dot_nt(x_ref[...], w_ref[...]) + b_ref[...]
    if act:
        if halves:
            bs = h.shape[0] // 2
            g = g_ref[...]
            bt = bt_ref[...]
            h = jnp.concatenate(
                [_bn(h[:bs], g, bt), _bn(h[bs:], g, bt)], axis=0)
        else:
            h = _bn(h, g_ref[...], bt_ref[...])
        h = _mish(h)
    o_ref[...] = h.astype(o_ref.dtype)


def _layer(x, W, b, g, beta, act, n_blk=None, halves=False,
           out_dtype=jnp.bfloat16):
    B, K = x.shape
    N = W.shape[0]
    if n_blk is None or n_blk > N:
        n_blk = N
    nn = N // n_blk
    if g is None:
        g = jnp.zeros((N,), jnp.float32)
        beta = jnp.zeros((N,), jnp.float32)
    body = functools.partial(_layer_body, act=act, halves=halves)
    return pl.pallas_call(
        body,
        grid=(nn,),
        in_specs=[
            pl.BlockSpec((B, K), lambda n: (0, 0)),
            pl.BlockSpec((n_blk, K), lambda n: (n, 0)),
            pl.BlockSpec((1, n_blk), lambda n: (0, n)),
            pl.BlockSpec((1, n_blk), lambda n: (0, n)),
            pl.BlockSpec((1, n_blk), lambda n: (0, n)),
        ],
        out_specs=pl.BlockSpec((B, n_blk), lambda n: (0, n)),
        out_shape=jax.ShapeDtypeStruct((B, N), out_dtype),
    )(x, W, b.reshape(1, N), g.reshape(1, N), beta.reshape(1, N))


def _final_body(x_ref, w_ref, b_ref, ty_ref, tx_ref, o_ref):
    n = pl.program_id(0)
    bs = ty_ref.shape[0]
    h = _dot_nt(x_ref[...], w_ref[...]) + b_ref[...]
    d0 = h[:bs] - ty_ref[...]
    d1 = h[bs:] - tx_ref[...]

    @pl.when(n == 0)
    def _init():
        o_ref[...] = jnp.zeros_like(o_ref)

    o_ref[...] += (jnp.sum(d0 * d0) + jnp.sum(d1 * d1)).reshape(1, 1)


def _final_layer_sse(x, W, b, t_y, t_x, n_blk=512):
    B2, K = x.shape
    B = B2 // 2
    N = W.shape[0]
    nn = N // n_blk
    return pl.pallas_call(
        _final_body,
        grid=(nn,),
        in_specs=[
            pl.BlockSpec((B2, K), lambda n: (0, 0)),
            pl.BlockSpec((n_blk, K), lambda n: (n, 0)),
            pl.BlockSpec((1, n_blk), lambda n: (0, n)),
            pl.BlockSpec((B, n_blk), lambda n: (0, n)),
            pl.BlockSpec((B, n_blk), lambda n: (0, n)),
        ],
        out_specs=pl.BlockSpec((1, 1), lambda n: (0, 0)),
        out_shape=jax.ShapeDtypeStruct((1, 1), jnp.float32),
    )(x, W, b.reshape(1, N), t_y, t_x)


def _vq_body(z_ref, z1_ref, cbx_ref, cby_ref, q_ref, s_ref):
    def one(z, cb):
        zz = jnp.sum(z * z, axis=1, keepdims=True)
        cc = jnp.sum(cb * cb, axis=1)[None, :]
        zc = _dot_nt(z.astype(jnp.bfloat16), cb.astype(jnp.bfloat16))
        d = zz + cc - 2.0 * zc
        dmin = jnp.min(d, axis=1, keepdims=True)
        ids = jax.lax.broadcasted_iota(jnp.int32, d.shape, 1)
        cand = jnp.where(d <= dmin, ids, d.shape[1])
        idx = jnp.min(cand, axis=1, keepdims=True)
        oh = (ids == idx).astype(jnp.float32)
        q = jax.lax.dot_general(
            oh, cb, (((1,), (0,)), ((), ())), preferred_element_type=jnp.float32)
        sse = jnp.sum((q - z) ** 2)
        return q, sse

    z = z_ref[...]
    z1 = z1_ref[...]
    q, sse_x = one(z, cbx_ref[...])
    q1, sse_y = one(z1, cby_ref[...])
    q_ref[...] = jnp.concatenate([q, q1], axis=0).astype(jnp.bfloat16)
    denom = z.shape[0] * z.shape[1]
    s_ref[...] = (((1.0 + _CC) * (sse_x + sse_y)
                   + _LZ * jnp.sum((z - z1) ** 2)) / denom).reshape(1, 1)


def _vq_both(z, z1, cb_x, cb_y):
    B, E = z.shape
    return pl.pallas_call(
        _vq_body,
        out_shape=(
            jax.ShapeDtypeStruct((2 * B, E), jnp.bfloat16),
            jax.ShapeDtypeStruct((1, 1), jnp.float32),
        ),
    )(z, z1, cb_x, cb_y)


def _encoder(inp, p):
    h = _layer(inp, p["W"][0], p["b"][0], p["g"][0], p["beta"][0], True,
               n_blk=256)
    h = _layer(h, p["W"][1], p["b"][1], p["g"][1], p["beta"][1], True,
               n_blk=256)
    h = _layer(h, p["W"][2], p["b"][2], p["g"][2], p["beta"][2], True)
    h = _layer(h, p["W"][3], p["b"][3], None, None, False, out_dtype=jnp.float32)
    return h


def _bf16_tree(p):
    return {
        "W": [w.astype(jnp.bfloat16) for w in p["W"]],
        "b": p["b"], "g": p["g"], "beta": p["beta"],
    }


def kernel(x, y, params):
    B, in_dim = x.shape
    px = _bf16_tree(params["enc_x"])
    py = _bf16_tree(params["enc_y"])
    pd = _bf16_tree(params["dec"])
    z = _encoder(x.astype(jnp.bfloat16), px)
    z1 = _encoder(y.astype(jnp.bfloat16), py)
    qc, s_vq = _vq_both(z, z1, params["cb_x"], params["cb_y"])
    h = _layer(qc, pd["W"][0], pd["b"][0], pd["g"][0], pd["beta"][0], True,
               halves=True)
    h = _layer(h, pd["W"][1], pd["b"][1], pd["g"][1], pd["beta"][1], True,
               halves=True)
    h = _layer(h, pd["W"][2], pd["b"][2], pd["g"][2], pd["beta"][2], True,
               n_blk=512, halves=True)
    sse = _final_layer_sse(h, pd["W"][3], pd["b"][3], y, x)
    recon = sse[0, 0] / (B * in_dim)
    return s_vq[0, 0] + recon


# speedup vs baseline: 1.2733x; 1.1110x over previous
import functools

import jax
import jax.numpy as jnp
from jax.experimental import pallas as pl
from jax.experimental.pallas import tpu as pltpu

_EPS = 1e-5
_CC = 0.25
_LZ = 10.0
_DV1 = 1.0
_DV2 = 1.0


def _dot_nt(a, b):
    return jax.lax.dot_general(
        a, b, (((1,), (1,)), ((), ())), preferred_element_type=jnp.float32)


def _mish(h):
    s = jnp.exp(jnp.minimum(h, 30.0))
    p = s * (s + 2.0)
    return h * p / (p + 2.0)


def _bn(h, g, beta):
    m = jnp.mean(h, axis=0, keepdims=True)
    v = jnp.maximum(jnp.mean(h * h, axis=0, keepdims=True) - m * m, 0.0)
    a = g / jnp.sqrt(v + _EPS)
    return h * a + (beta - m * a)


def _bn2(h, g, beta):
    bs = h.shape[0] // 2
    return jnp.concatenate([_bn(h[:bs], g, beta), _bn(h[bs:], g, beta)], axis=0)


def _enc_body(x_ref, w1_ref, b1_ref, g1_ref, t1_ref,
              w2_ref, b2_ref, g2_ref, t2_ref,
              w3_ref, b3_ref, g3_ref, t3_ref,
              w4_ref, b4_ref, z_ref, h1_ref, *, nn, n_blk):
    n = pl.program_id(0)
    h = _dot_nt(x_ref[...], w1_ref[...]) + b1_ref[...]
    h = _mish(_bn(h, g1_ref[...], t1_ref[...]))
    h1_ref[:, pl.ds(n * n_blk, n_blk)] = h.astype(jnp.bfloat16)

    @pl.when(n == nn - 1)
    def _tail():
        h2 = _mish(_bn(_dot_nt(h1_ref[...], w2_ref[...]) + b2_ref[...],
                       g2_ref[...], t2_ref[...])).astype(jnp.bfloat16)
        h3 = _mish(_bn(_dot_nt(h2, w3_ref[...]) + b3_ref[...],
                       g3_ref[...], t3_ref[...])).astype(jnp.bfloat16)
        z_ref[...] = _dot_nt(h3, w4_ref[...]) + b4_ref[...]


def _encoder(x, p, n_blk=256):
    B, K = x.shape
    W1, W2, W3, W4 = p["W"]
    N1 = W1.shape[0]
    nn = N1 // n_blk
    ED = W4.shape[0]

    def row(v):
        return v.reshape(1, -1)

    const = lambda i, j: (lambda n: (i, j))
    body = functools.partial(_enc_body, nn=nn, n_blk=n_blk)
    return pl.pallas_call(
        body,
        grid=(nn,),
        in_specs=[
            pl.BlockSpec((B, K), lambda n: (0, 0)),
            pl.BlockSpec((n_blk, K), lambda n: (n, 0)),
            pl.BlockSpec((1, n_blk), lambda n: (0, n)),
            pl.BlockSpec((1, n_blk), lambda n: (0, n)),
            pl.BlockSpec((1, n_blk), lambda n: (0, n)),
            pl.BlockSpec(W2.shape, const(0, 0)),
            pl.BlockSpec((1, W2.shape[0]), const(0, 0)),
            pl.BlockSpec((1, W2.shape[0]), const(0, 0)),
            pl.BlockSpec((1, W2.shape[0]), const(0, 0)),
            pl.BlockSpec(W3.shape, const(0, 0)),
            pl.BlockSpec((1, W3.shape[0]), const(0, 0)),
            pl.BlockSpec((1, W3.shape[0]), const(0, 0)),
            pl.BlockSpec((1, W3.shape[0]), const(0, 0)),
            pl.BlockSpec(W4.shape, const(0, 0)),
            pl.BlockSpec((1, ED), const(0, 0)),
        ],
        out_specs=pl.BlockSpec((B, ED), lambda n: (0, 0)),
        out_shape=jax.ShapeDtypeStruct((B, ED), jnp.float32),
        scratch_shapes=[pltpu.VMEM((B, N1), jnp.bfloat16)],
    )(x, W1, row(p["b"][0]), row(p["g"][0]), row(p["beta"][0]),
      W2, row(p["b"][1]), row(p["g"][1]), row(p["beta"][1]),
      W3, row(p["b"][2]), row(p["g"][2]), row(p["beta"][2]),
      W4, row(p["b"][3]))


def _vq_body(z_ref, z1_ref, cbx_ref, cby_ref, q_ref, s_ref):
    def one(z, cb):
        zz = jnp.sum(z * z, axis=1, keepdims=True)
        cc = jnp.sum(cb * cb, axis=1)[None, :]
        zc = _dot_nt(z.astype(jnp.bfloat16), cb.astype(jnp.bfloat16))
        d = zz + cc - 2.0 * zc
        dmin = jnp.min(d, axis=1, keepdims=True)
        ids = jax.lax.broadcasted_iota(jnp.int32, d.shape, 1)
        cand = jnp.where(d <= dmin, ids, d.shape[1])
        idx = jnp.min(cand, axis=1, keepdims=True)
        oh = (ids == idx).astype(jnp.float32)
        q = jax.lax.dot_general(
            oh, cb, (((1,), (0,)), ((), ())), preferred_element_type=jnp.float32)
        sse = jnp.sum((q - z) ** 2)
        return q, sse

    z = z_ref[...]
    z1 = z1_ref[...]
    q, sse_x = one(z, cbx_ref[...])
    q1, sse_y = one(z1, cby_ref[...])
    q_ref[...] = jnp.concatenate([q, q1], axis=0).astype(jnp.bfloat16)
    denom = z.shape[0] * z.shape[1]
    s_ref[...] = (((1.0 + _CC) * (sse_x + sse_y)
                   + _LZ * jnp.sum((z - z1) ** 2)) / denom).reshape(1, 1)


def _vq_both(z, z1, cb_x, cb_y):
    B, E = z.shape
    return pl.pallas_call(
        _vq_body,
        out_shape=(
            jax.ShapeDtypeStruct((2 * B, E), jnp.bfloat16),
            jax.ShapeDtypeStruct((1, 1), jnp.float32),
        ),
    )(z, z1, cb_x, cb_y)


def _dec123_body(q_ref, w1_ref, b1_ref, g1_ref, t1_ref,
                 w2_ref, b2_ref, g2_ref, t2_ref,
                 w3_ref, b3_ref, g3_ref, t3_ref, o_ref):
    h1 = _mish(_bn2(_dot_nt(q_ref[...], w1_ref[...]) + b1_ref[...],
                    g1_ref[...], t1_ref[...])).astype(jnp.bfloat16)
    h2 = _mish(_bn2(_dot_nt(h1, w2_ref[...]) + b2_ref[...],
                    g2_ref[...], t2_ref[...])).astype(jnp.bfloat16)
    h3 = _mish(_bn2(_dot_nt(h2, w3_ref[...]) + b3_ref[...],
                    g3_ref[...], t3_ref[...]))
    o_ref[...] = h3.astype(jnp.bfloat16)


def _dec123(qc, p):
    B2 = qc.shape[0]
    W1, W2, W3 = p["W"][0], p["W"][1], p["W"][2]
    N3 = W3.shape[0]

    def row(v):
        return v.reshape(1, -1)

    return pl.pallas_call(
        _dec123_body,
        out_shape=jax.ShapeDtypeStruct((B2, N3), jnp.bfloat16),
    )(qc, W1, row(p["b"][0]), row(p["g"][0]), row(p["beta"][0]),
      W2, row(p["b"][1]), row(p["g"][1]), row(p["beta"][1]),
      W3, row(p["b"][2]), row(p["g"][2]), row(p["beta"][2]))


def _final_body(x_ref, w_ref, b_ref, ty_ref, tx_ref, o_ref):
    n = pl.program_id(0)
    bs = ty_ref.shape[0]
    h = _dot_nt(x_ref[...], w_ref[...]) + b_ref[...]
    d0 = h[:bs] - ty_ref[...]
    d1 = h[bs:] - tx_ref[...]

    @pl.when(n == 0)
    def _init():
        o_ref[...] = jnp.zeros_like(o_ref)

    o_ref[...] += (jnp.sum(d0 * d0) + jnp.sum(d1 * d1)).reshape(1, 1)


def _final_layer_sse(x, W, b, t_y, t_x, n_blk=512):
    B2, K = x.shape
    B = B2 // 2
    N = W.shape[0]
    nn = N // n_blk
    return pl.pallas_call(
        _final_body,
        grid=(nn,),
        in_specs=[
            pl.BlockSpec((B2, K), lambda n: (0, 0)),
            pl.BlockSpec((n_blk, K), lambda n: (n, 0)),
            pl.BlockSpec((1, n_blk), lambda n: (0, n)),
            pl.BlockSpec((B, n_blk), lambda n: (0, n)),
            pl.BlockSpec((B, n_blk), lambda n: (0, n)),
        ],
        out_specs=pl.BlockSpec((1, 1), lambda n: (0, 0)),
        out_shape=jax.ShapeDtypeStruct((1, 1), jnp.float32),
    )(x, W, b.reshape(1, N), t_y, t_x)


def _bf16_tree(p):
    return {
        "W": [w.astype(jnp.bfloat16) for w in p["W"]],
        "b": p["b"], "g": p["g"], "beta": p["beta"],
    }


def kernel(x, y, params):
    B, in_dim = x.shape
    px = _bf16_tree(params["enc_x"])
    py = _bf16_tree(params["enc_y"])
    pd = _bf16_tree(params["dec"])
    z = _encoder(x.astype(jnp.bfloat16), px)
    z1 = _encoder(y.astype(jnp.bfloat16), py)
    qc, s_vq = _vq_both(z, z1, params["cb_x"], params["cb_y"])
    h3 = _dec123(qc, pd)
    sse = _final_layer_sse(h3, pd["W"][3], pd["b"][3], y, x)
    recon = sse[0, 0] / (B * in_dim)
    return s_vq[0, 0] + recon


# speedup vs baseline: 1.3502x; 1.0604x over previous
import functools

import jax
import jax.numpy as jnp
from jax.experimental import pallas as pl
from jax.experimental.pallas import tpu as pltpu

_EPS = 1e-5
_CC = 0.25
_LZ = 10.0
_DV1 = 1.0
_DV2 = 1.0


def _dot_nt(a, b):
    return jax.lax.dot_general(
        a, b, (((1,), (1,)), ((), ())), preferred_element_type=jnp.float32)


def _mish16(x):
    s = jnp.exp2(jnp.minimum(x, jnp.bfloat16(60.0)) * jnp.bfloat16(1.4426950))
    p = s * (s + jnp.bfloat16(2.0))
    return x * p / (p + jnp.bfloat16(2.0))


def _bn_mish(h, g, beta):
    m = jnp.mean(h, axis=0, keepdims=True)
    v = jnp.maximum(jnp.mean(h * h, axis=0, keepdims=True) - m * m, 0.0)
    a = g / jnp.sqrt(v + _EPS)
    b = beta - m * a
    return _mish16(h.astype(jnp.bfloat16) * a.astype(jnp.bfloat16)
                   + b.astype(jnp.bfloat16))


def _bn_mish2(h, g, beta):
    bs = h.shape[0] // 2
    return jnp.concatenate(
        [_bn_mish(h[:bs], g, beta), _bn_mish(h[bs:], g, beta)], axis=0)


def _enc_body(x_ref, w1_ref, b1_ref, g1_ref, t1_ref,
              w2_ref, b2_ref, g2_ref, t2_ref,
              w3_ref, b3_ref, g3_ref, t3_ref,
              w4_ref, b4_ref, z_ref, h1_ref, *, nn, n_blk):
    n = pl.program_id(0)
    h = _dot_nt(x_ref[...], w1_ref[...]) + b1_ref[...]
    h1_ref[:, pl.ds(n * n_blk, n_blk)] = _bn_mish(h, g1_ref[...], t1_ref[...])

    @pl.when(n == nn - 1)
    def _tail():
        h2 = _bn_mish(_dot_nt(h1_ref[...], w2_ref[...]) + b2_ref[...],
                      g2_ref[...], t2_ref[...])
        h3 = _bn_mish(_dot_nt(h2, w3_ref[...]) + b3_ref[...],
                      g3_ref[...], t3_ref[...])
        z_ref[...] = _dot_nt(h3, w4_ref[...]) + b4_ref[...]


def _encoder(x, p, n_blk=256):
    B, K = x.shape
    W1, W2, W3, W4 = p["W"]
    N1 = W1.shape[0]
    nn = N1 // n_blk
    ED = W4.shape[0]

    def row(v):
        return v.reshape(1, -1)

    const = lambda i, j: (lambda n: (i, j))
    body = functools.partial(_enc_body, nn=nn, n_blk=n_blk)
    return pl.pallas_call(
        body,
        grid=(nn,),
        in_specs=[
            pl.BlockSpec((B, K), lambda n: (0, 0)),
            pl.BlockSpec((n_blk, K), lambda n: (n, 0)),
            pl.BlockSpec((1, n_blk), lambda n: (0, n)),
            pl.BlockSpec((1, n_blk), lambda n: (0, n)),
            pl.BlockSpec((1, n_blk), lambda n: (0, n)),
            pl.BlockSpec(W2.shape, const(0, 0)),
            pl.BlockSpec((1, W2.shape[0]), const(0, 0)),
            pl.BlockSpec((1, W2.shape[0]), const(0, 0)),
            pl.BlockSpec((1, W2.shape[0]), const(0, 0)),
            pl.BlockSpec(W3.shape, const(0, 0)),
            pl.BlockSpec((1, W3.shape[0]), const(0, 0)),
            pl.BlockSpec((1, W3.shape[0]), const(0, 0)),
            pl.BlockSpec((1, W3.shape[0]), const(0, 0)),
            pl.BlockSpec(W4.shape, const(0, 0)),
            pl.BlockSpec((1, ED), const(0, 0)),
        ],
        out_specs=pl.BlockSpec((B, ED), lambda n: (0, 0)),
        out_shape=jax.ShapeDtypeStruct((B, ED), jnp.float32),
        scratch_shapes=[pltpu.VMEM((B, N1), jnp.bfloat16)],
    )(x, W1, row(p["b"][0]), row(p["g"][0]), row(p["beta"][0]),
      W2, row(p["b"][1]), row(p["g"][1]), row(p["beta"][1]),
      W3, row(p["b"][2]), row(p["g"][2]), row(p["beta"][2]),
      W4, row(p["b"][3]))


def _vq_body(z_ref, z1_ref, cbx_ref, cby_ref, q_ref, s_ref):
    def one(z, cb):
        zz = jnp.sum(z * z, axis=1, keepdims=True)
        cc = jnp.sum(cb * cb, axis=1)[None, :]
        zc = _dot_nt(z.astype(jnp.bfloat16), cb.astype(jnp.bfloat16))
        d = zz + cc - 2.0 * zc
        dmin = jnp.min(d, axis=1, keepdims=True)
        ids = jax.lax.broadcasted_iota(jnp.int32, d.shape, 1)
        cand = jnp.where(d <= dmin, ids, d.shape[1])
        idx = jnp.min(cand, axis=1, keepdims=True)
        oh = (ids == idx).astype(jnp.float32)
        q = jax.lax.dot_general(
            oh, cb, (((1,), (0,)), ((), ())), preferred_element_type=jnp.float32)
        sse = jnp.sum((q - z) ** 2)
        return q, sse

    z = z_ref[...]
    z1 = z1_ref[...]
    q, sse_x = one(z, cbx_ref[...])
    q1, sse_y = one(z1, cby_ref[...])
    q_ref[...] = jnp.concatenate([q, q1], axis=0).astype(jnp.bfloat16)
    denom = z.shape[0] * z.shape[1]
    s_ref[...] = (((1.0 + _CC) * (sse_x + sse_y)
                   + _LZ * jnp.sum((z - z1) ** 2)) / denom).reshape(1, 1)


def _vq_both(z, z1, cb_x, cb_y):
    B, E = z.shape
    return pl.pallas_call(
        _vq_body,
        out_shape=(
            jax.ShapeDtypeStruct((2 * B, E), jnp.bfloat16),
            jax.ShapeDtypeStruct((1, 1), jnp.float32),
        ),
    )(z, z1, cb_x, cb_y)


def _dec123_body(q_ref, w1_ref, b1_ref, g1_ref, t1_ref,
                 w2_ref, b2_ref, g2_ref, t2_ref,
                 w3_ref, b3_ref, g3_ref, t3_ref, o_ref):
    h1 = _bn_mish2(_dot_nt(q_ref[...], w1_ref[...]) + b1_ref[...],
                   g1_ref[...], t1_ref[...])
    h2 = _bn_mish2(_dot_nt(h1, w2_ref[...]) + b2_ref[...],
                   g2_ref[...], t2_ref[...])
    o_ref[...] = _bn_mish2(_dot_nt(h2, w3_ref[...]) + b3_ref[...],
                           g3_ref[...], t3_ref[...])


def _dec123(qc, p):
    B2 = qc.shape[0]
    W1, W2, W3 = p["W"][0], p["W"][1], p["W"][2]
    N3 = W3.shape[0]

    def row(v):
        return v.reshape(1, -1)

    return pl.pallas_call(
        _dec123_body,
        out_shape=jax.ShapeDtypeStruct((B2, N3), jnp.bfloat16),
    )(qc, W1, row(p["b"][0]), row(p["g"][0]), row(p["beta"][0]),
      W2, row(p["b"][1]), row(p["g"][1]), row(p["beta"][1]),
      W3, row(p["b"][2]), row(p["g"][2]), row(p["beta"][2]))


def _final_body(x_ref, w_ref, b_ref, ty_ref, tx_ref, o_ref):
    n = pl.program_id(0)
    bs = ty_ref.shape[0]
    h = _dot_nt(x_ref[...], w_ref[...]) + b_ref[...]
    d0 = h[:bs] - ty_ref[...]
    d1 = h[bs:] - tx_ref[...]

    @pl.when(n == 0)
    def _init():
        o_ref[...] = jnp.zeros_like(o_ref)

    o_ref[...] += (jnp.sum(d0 * d0) + jnp.sum(d1 * d1)).reshape(1, 1)


def _final_layer_sse(x, W, b, t_y, t_x, n_blk=512):
    B2, K = x.shape
    B = B2 // 2
    N = W.shape[0]
    nn = N // n_blk
    return pl.pallas_call(
        _final_body,
        grid=(nn,),
        in_specs=[
            pl.BlockSpec((B2, K), lambda n: (0, 0)),
            pl.BlockSpec((n_blk, K), lambda n: (n, 0)),
            pl.BlockSpec((1, n_blk), lambda n: (0, n)),
            pl.BlockSpec((B, n_blk), lambda n: (0, n)),
            pl.BlockSpec((B, n_blk), lambda n: (0, n)),
        ],
        out_specs=pl.BlockSpec((1, 1), lambda n: (0, 0)),
        out_shape=jax.ShapeDtypeStruct((1, 1), jnp.float32),
    )(x, W, b.reshape(1, N), t_y, t_x)


def _bf16_tree(p):
    return {
        "W": [w.astype(jnp.bfloat16) for w in p["W"]],
        "b": p["b"], "g": p["g"], "beta": p["beta"],
    }


def kernel(x, y, params):
    B, in_dim = x.shape
    px = _bf16_tree(params["enc_x"])
    py = _bf16_tree(params["enc_y"])
    pd = _bf16_tree(params["dec"])
    z = _encoder(x.astype(jnp.bfloat16), px)
    z1 = _encoder(y.astype(jnp.bfloat16), py)
    qc, s_vq = _vq_both(z, z1, params["cb_x"], params["cb_y"])
    h3 = _dec123(qc, pd)
    sse = _final_layer_sse(h3, pd["W"][3], pd["b"][3], y, x)
    recon = sse[0, 0] / (B * in_dim)
    return s_vq[0, 0] + recon


# speedup vs baseline: 1.6241x; 1.2028x over previous
import functools

import jax
import jax.numpy as jnp
from jax.experimental import pallas as pl
from jax.experimental.pallas import tpu as pltpu

_EPS = 1e-5
_CC = 0.25
_LZ = 10.0
_DV1 = 1.0
_DV2 = 1.0


def _dot_nt(a, b):
    return jax.lax.dot_general(
        a.astype(jnp.bfloat16), b.astype(jnp.bfloat16),
        (((1,), (1,)), ((), ())), preferred_element_type=jnp.float32)


def _mish16(x):
    s = jnp.exp2(jnp.minimum(x, jnp.bfloat16(60.0)) * jnp.bfloat16(1.4426950))
    p = s * (s + jnp.bfloat16(2.0))
    return x * p / (p + jnp.bfloat16(2.0))


def _bn_mish(h, g, beta):
    m = jnp.mean(h, axis=0, keepdims=True)
    v = jnp.maximum(jnp.mean(h * h, axis=0, keepdims=True) - m * m, 0.0)
    a = g / jnp.sqrt(v + _EPS)
    b = beta - m * a
    return _mish16(h.astype(jnp.bfloat16) * a.astype(jnp.bfloat16)
                   + b.astype(jnp.bfloat16))


def _bn_mish2(h, g, beta):
    bs = h.shape[0] // 2
    return jnp.concatenate(
        [_bn_mish(h[:bs], g, beta), _bn_mish(h[bs:], g, beta)], axis=0)


def _enc_body(x_ref, w1_ref, b1_ref, g1_ref, t1_ref,
              w2_ref, b2_ref, g2_ref, t2_ref,
              w3_ref, b3_ref, g3_ref, t3_ref,
              w4_ref, b4_ref, z_ref, h1_ref, *, nn, n_blk):
    n = pl.program_id(0)
    h = _dot_nt(x_ref[...], w1_ref[...]) + b1_ref[...]
    h1_ref[:, pl.ds(n * n_blk, n_blk)] = _bn_mish(h, g1_ref[...], t1_ref[...])

    @pl.when(n == nn - 1)
    def _tail():
        h2 = _bn_mish(_dot_nt(h1_ref[...], w2_ref[...]) + b2_ref[...],
                      g2_ref[...], t2_ref[...])
        h3 = _bn_mish(_dot_nt(h2, w3_ref[...]) + b3_ref[...],
                      g3_ref[...], t3_ref[...])
        z_ref[...] = _dot_nt(h3, w4_ref[...]) + b4_ref[...]


def _encoder(x, p, n_blk=256):
    B, K = x.shape
    W1, W2, W3, W4 = p["W"]
    N1 = W1.shape[0]
    nn = N1 // n_blk
    ED = W4.shape[0]

    def row(v):
        return v.reshape(1, -1)

    const = lambda i, j: (lambda n: (i, j))
    body = functools.partial(_enc_body, nn=nn, n_blk=n_blk)
    return pl.pallas_call(
        body,
        grid=(nn,),
        in_specs=[
            pl.BlockSpec((B, K), lambda n: (0, 0)),
            pl.BlockSpec((n_blk, K), lambda n: (n, 0)),
            pl.BlockSpec((1, n_blk), lambda n: (0, n)),
            pl.BlockSpec((1, n_blk), lambda n: (0, n)),
            pl.BlockSpec((1, n_blk), lambda n: (0, n)),
            pl.BlockSpec(W2.shape, const(0, 0)),
            pl.BlockSpec((1, W2.shape[0]), const(0, 0)),
            pl.BlockSpec((1, W2.shape[0]), const(0, 0)),
            pl.BlockSpec((1, W2.shape[0]), const(0, 0)),
            pl.BlockSpec(W3.shape, const(0, 0)),
            pl.BlockSpec((1, W3.shape[0]), const(0, 0)),
            pl.BlockSpec((1, W3.shape[0]), const(0, 0)),
            pl.BlockSpec((1, W3.shape[0]), const(0, 0)),
            pl.BlockSpec(W4.shape, const(0, 0)),
            pl.BlockSpec((1, ED), const(0, 0)),
        ],
        out_specs=pl.BlockSpec((B, ED), lambda n: (0, 0)),
        out_shape=jax.ShapeDtypeStruct((B, ED), jnp.float32),
        scratch_shapes=[pltpu.VMEM((B, N1), jnp.bfloat16)],
    )(x, W1, row(p["b"][0]), row(p["g"][0]), row(p["beta"][0]),
      W2, row(p["b"][1]), row(p["g"][1]), row(p["beta"][1]),
      W3, row(p["b"][2]), row(p["g"][2]), row(p["beta"][2]),
      W4, row(p["b"][3]))


def _vq_body(z_ref, z1_ref, cbx_ref, cby_ref, q_ref, s_ref):
    def one(z, cb):
        zz = jnp.sum(z * z, axis=1, keepdims=True)
        cc = jnp.sum(cb * cb, axis=1)[None, :]
        zc = _dot_nt(z, cb)
        d = zz + cc - 2.0 * zc
        dmin = jnp.min(d, axis=1, keepdims=True)
        ids = jax.lax.broadcasted_iota(jnp.int32, d.shape, 1)
        cand = jnp.where(d <= dmin, ids, d.shape[1])
        idx = jnp.min(cand, axis=1, keepdims=True)
        oh = (ids == idx).astype(jnp.float32)
        q = jax.lax.dot_general(
            oh, cb, (((1,), (0,)), ((), ())), preferred_element_type=jnp.float32)
        sse = jnp.sum((q - z) ** 2)
        return q, sse

    z = z_ref[...]
    z1 = z1_ref[...]
    q, sse_x = one(z, cbx_ref[...])
    q1, sse_y = one(z1, cby_ref[...])
    q_ref[...] = jnp.concatenate([q, q1], axis=0).astype(jnp.bfloat16)
    denom = z.shape[0] * z.shape[1]
    s_ref[...] = (((1.0 + _CC) * (sse_x + sse_y)
                   + _LZ * jnp.sum((z - z1) ** 2)) / denom).reshape(1, 1)


def _vq_both(z, z1, cb_x, cb_y):
    B, E = z.shape
    return pl.pallas_call(
        _vq_body,
        out_shape=(
            jax.ShapeDtypeStruct((2 * B, E), jnp.bfloat16),
            jax.ShapeDtypeStruct((1, 1), jnp.float32),
        ),
    )(z, z1, cb_x, cb_y)


def _dec_body(q_ref, svq_ref,
              w1_ref, b1_ref, g1_ref, t1_ref,
              w2_ref, b2_ref, g2_ref, t2_ref,
              w3_ref, b3_ref, g3_ref, t3_ref,
              w4_ref, b4_ref, ty_ref, tx_ref,
              o_ref, h3_ref, *, scale):
    n = pl.program_id(0)

    @pl.when(n == 0)
    def _head():
        h1 = _bn_mish2(_dot_nt(q_ref[...], w1_ref[...]) + b1_ref[...],
                       g1_ref[...], t1_ref[...])
        h2 = _bn_mish2(_dot_nt(h1, w2_ref[...]) + b2_ref[...],
                       g2_ref[...], t2_ref[...])
        h3_ref[...] = _bn_mish2(_dot_nt(h2, w3_ref[...]) + b3_ref[...],
                                g3_ref[...], t3_ref[...])
        o_ref[...] = svq_ref[...]

    bs = ty_ref.shape[0]
    h = _dot_nt(h3_ref[...], w4_ref[...]) + b4_ref[...]
    d0 = h[:bs] - ty_ref[...]
    d1 = h[bs:] - tx_ref[...]
    o_ref[...] += ((jnp.sum(d0 * d0) / _DV1 + jnp.sum(d1 * d1) / _DV2)
                   * scale).reshape(1, 1)


def _decoder_loss(qc, s_vq, p, t_y, t_x, n_blk=512):
    B2 = qc.shape[0]
    B, in_dim = t_y.shape
    W1, W2, W3, W4 = p["W"]
    nn = W4.shape[0] // n_blk

    def row(v):
        return v.reshape(1, -1)

    const = lambda i, j: (lambda n: (i, j))
    body = functools.partial(_dec_body, scale=1.0 / (B * in_dim))
    return pl.pallas_call(
        body,
        grid=(nn,),
        in_specs=[
            pl.BlockSpec((B2, W1.shape[1]), const(0, 0)),
            pl.BlockSpec((1, 1), const(0, 0)),
            pl.BlockSpec(W1.shape, const(0, 0)),
            pl.BlockSpec((1, W1.shape[0]), const(0, 0)),
            pl.BlockSpec((1, W1.shape[0]), const(0, 0)),
            pl.BlockSpec((1, W1.shape[0]), const(0, 0)),
            pl.BlockSpec(W2.shape, const(0, 0)),
            pl.BlockSpec((1, W2.shape[0]), const(0, 0)),
            pl.BlockSpec((1, W2.shape[0]), const(0, 0)),
            pl.BlockSpec((1, W2.shape[0]), const(0, 0)),
            pl.BlockSpec(W3.shape, const(0, 0)),
            pl.BlockSpec((1, W3.shape[0]), const(0, 0)),
            pl.BlockSpec((1, W3.shape[0]), const(0, 0)),
            pl.BlockSpec((1, W3.shape[0]), const(0, 0)),
            pl.BlockSpec((n_blk, W4.shape[1]), lambda n: (n, 0)),
            pl.BlockSpec((1, n_blk), lambda n: (0, n)),
            pl.BlockSpec((B, n_blk), lambda n: (0, n)),
            pl.BlockSpec((B, n_blk), lambda n: (0, n)),
        ],
        out_specs=pl.BlockSpec((1, 1), lambda n: (0, 0)),
        out_shape=jax.ShapeDtypeStruct((1, 1), jnp.float32),
        scratch_shapes=[pltpu.VMEM((B2, W3.shape[0]), jnp.bfloat16)],
    )(qc, s_vq,
      W1, row(p["b"][0]), row(p["g"][0]), row(p["beta"][0]),
      W2, row(p["b"][1]), row(p["g"][1]), row(p["beta"][1]),
      W3, row(p["b"][2]), row(p["g"][2]), row(p["beta"][2]),
      W4, row(p["b"][3]), t_y, t_x)


def kernel(x, y, params):
    z = _encoder(x.astype(jnp.bfloat16), params["enc_x"])
    z1 = _encoder(y.astype(jnp.bfloat16), params["enc_y"])
    qc, s_vq = _vq_both(z, z1, params["cb_x"], params["cb_y"])
    total = _decoder_loss(qc, s_vq, params["dec"], y, x)
    return total[0, 0]


# speedup vs baseline: 1.9025x; 1.1714x over previous
import functools

import jax
import jax.numpy as jnp
from jax.experimental import pallas as pl
from jax.experimental.pallas import tpu as pltpu

_EPS = 1e-5
_CC = 0.25
_LZ = 10.0
_DV1 = 1.0
_DV2 = 1.0


def _dot_nt(a, b):
    return jax.lax.dot_general(
        a.astype(jnp.bfloat16), b.astype(jnp.bfloat16),
        (((1,), (1,)), ((), ())), preferred_element_type=jnp.float32)


def _mish16(x):
    s = jnp.exp2(jnp.minimum(x, jnp.bfloat16(60.0)) * jnp.bfloat16(1.4426950))
    p = s * (s + jnp.bfloat16(2.0))
    return x * p / (p + jnp.bfloat16(2.0))


def _bn_mish(h, g, beta):
    m = jnp.mean(h, axis=0, keepdims=True)
    v = jnp.maximum(jnp.mean(h * h, axis=0, keepdims=True) - m * m, 0.0)
    a = g / jnp.sqrt(v + _EPS)
    b = beta - m * a
    return _mish16(h.astype(jnp.bfloat16) * a.astype(jnp.bfloat16)
                   + b.astype(jnp.bfloat16))


def _bn_mish2(h, g, beta):
    bs = h.shape[0] // 2
    return jnp.concatenate(
        [_bn_mish(h[:bs], g, beta), _bn_mish(h[bs:], g, beta)], axis=0)


def _enc_body(x_ref, w1_ref, b1_ref, g1_ref, t1_ref,
              w2_ref, b2_ref, g2_ref, t2_ref,
              w3_ref, b3_ref, g3_ref, t3_ref,
              w4_ref, b4_ref, z_ref, h1_ref, s_ref, s2_ref, *, nb, b_blk):
    n = pl.program_id(0)
    h = _dot_nt(x_ref[...], w1_ref[...]) + b1_ref[...]
    h1_ref[pl.ds(n * b_blk, b_blk), :] = h
    s = jnp.sum(h, axis=0, keepdims=True)
    s2 = jnp.sum(h * h, axis=0, keepdims=True)

    @pl.when(n == 0)
    def _init():
        s_ref[...] = s
        s2_ref[...] = s2

    @pl.when(n > 0)
    def _acc():
        s_ref[...] += s
        s2_ref[...] += s2

    @pl.when(n == nb - 1)
    def _tail():
        B = nb * b_blk
        m = s_ref[...] / B
        v = jnp.maximum(s2_ref[...] / B - m * m, 0.0)
        a = g1_ref[...] / jnp.sqrt(v + _EPS)
        b = t1_ref[...] - m * a
        h1b = _mish16(h1_ref[...].astype(jnp.bfloat16)
                      * a.astype(jnp.bfloat16) + b.astype(jnp.bfloat16))
        h2 = _bn_mish(_dot_nt(h1b, w2_ref[...]) + b2_ref[...],
                      g2_ref[...], t2_ref[...])
        h3 = _bn_mish(_dot_nt(h2, w3_ref[...]) + b3_ref[...],
                      g3_ref[...], t3_ref[...])
        z_ref[...] = _dot_nt(h3, w4_ref[...]) + b4_ref[...]


def _encoder(x, p, b_blk=512):
    B, K = x.shape
    W1, W2, W3, W4 = p["W"]
    N1 = W1.shape[0]
    nb = B // b_blk
    ED = W4.shape[0]

    def row(v):
        return v.reshape(1, -1)

    const = lambda i, j: (lambda n: (i, j))
    body = functools.partial(_enc_body, nb=nb, b_blk=b_blk)
    return pl.pallas_call(
        body,
        grid=(nb,),
        in_specs=[
            pl.BlockSpec((b_blk, K), lambda n: (n, 0)),
            pl.BlockSpec(W1.shape, const(0, 0)),
            pl.BlockSpec((1, N1), const(0, 0)),
            pl.BlockSpec((1, N1), const(0, 0)),
            pl.BlockSpec((1, N1), const(0, 0)),
            pl.BlockSpec(W2.shape, const(0, 0)),
            pl.BlockSpec((1, W2.shape[0]), const(0, 0)),
            pl.BlockSpec((1, W2.shape[0]), const(0, 0)),
            pl.BlockSpec((1, W2.shape[0]), const(0, 0)),
            pl.BlockSpec(W3.shape, const(0, 0)),
            pl.BlockSpec((1, W3.shape[0]), const(0, 0)),
            pl.BlockSpec((1, W3.shape[0]), const(0, 0)),
            pl.BlockSpec((1, W3.shape[0]), const(0, 0)),
            pl.BlockSpec(W4.shape, const(0, 0)),
            pl.BlockSpec((1, ED), const(0, 0)),
        ],
        out_specs=pl.BlockSpec((B, ED), lambda n: (0, 0)),
        out_shape=jax.ShapeDtypeStruct((B, ED), jnp.float32),
        scratch_shapes=[pltpu.VMEM((B, N1), jnp.float32),
                        pltpu.VMEM((1, N1), jnp.float32),
                        pltpu.VMEM((1, N1), jnp.float32)],
    )(x, W1, row(p["b"][0]), row(p["g"][0]), row(p["beta"][0]),
      W2, row(p["b"][1]), row(p["g"][1]), row(p["beta"][1]),
      W3, row(p["b"][2]), row(p["g"][2]), row(p["beta"][2]),
      W4, row(p["b"][3]))


def _vq_one(z, cb):
    zz = jnp.sum(z * z, axis=1, keepdims=True)
    cc = jnp.sum(cb * cb, axis=1)[None, :]
    zc = _dot_nt(z, cb)
    d = zz + cc - 2.0 * zc
    dmin = jnp.min(d, axis=1, keepdims=True)
    ids = jax.lax.broadcasted_iota(jnp.int32, d.shape, 1)
    cand = jnp.where(d <= dmin, ids, d.shape[1])
    idx = jnp.min(cand, axis=1, keepdims=True)
    oh = (ids == idx).astype(jnp.float32)
    q = jax.lax.dot_general(
        oh, cb, (((1,), (0,)), ((), ())), preferred_element_type=jnp.float32)
    sse = jnp.sum((q - z) ** 2)
    return q, sse


def _dec_body(z_ref, z1_ref, cbx_ref, cby_ref,
              w1_ref, b1_ref, g1_ref, t1_ref,
              w2_ref, b2_ref, g2_ref, t2_ref,
              w3_ref, b3_ref, g3_ref, t3_ref,
              w4_ref, b4_ref, ty_ref, tx_ref,
              o_ref, h3_ref, *, scale, vq_denom):
    n = pl.program_id(0)

    @pl.when(n == 0)
    def _head():
        z = z_ref[...]
        z1 = z1_ref[...]
        q, sse_x = _vq_one(z, cbx_ref[...])
        q1, sse_y = _vq_one(z1, cby_ref[...])
        s_vq = ((1.0 + _CC) * (sse_x + sse_y)
                + _LZ * jnp.sum((z - z1) ** 2)) / vq_denom
        qc = jnp.concatenate([q, q1], axis=0).astype(jnp.bfloat16)
        h1 = _bn_mish2(_dot_nt(qc, w1_ref[...]) + b1_ref[...],
                       g1_ref[...], t1_ref[...])
        h2 = _bn_mish2(_dot_nt(h1, w2_ref[...]) + b2_ref[...],
                       g2_ref[...], t2_ref[...])
        h3_ref[...] = _bn_mish2(_dot_nt(h2, w3_ref[...]) + b3_ref[...],
                                g3_ref[...], t3_ref[...])
        o_ref[...] = s_vq.reshape(1, 1)

    bs = ty_ref.shape[0]
    h = _dot_nt(h3_ref[...], w4_ref[...]) + b4_ref[...]
    d0 = h[:bs] - ty_ref[...]
    d1 = h[bs:] - tx_ref[...]
    o_ref[...] += ((jnp.sum(d0 * d0) / _DV1 + jnp.sum(d1 * d1) / _DV2)
                   * scale).reshape(1, 1)


def _decoder_loss(z, z1, cb_x, cb_y, p, t_y, t_x, n_blk=256):
    B, in_dim = t_y.shape
    B2 = 2 * B
    W1, W2, W3, W4 = p["W"]
    nn = W4.shape[0] // n_blk

    def row(v):
        return v.reshape(1, -1)

    const = lambda i, j: (lambda n: (i, j))
    body = functools.partial(_dec_body, scale=1.0 / (B * in_dim),
                             vq_denom=float(B * z.shape[1]))
    return pl.pallas_call(
        body,
        grid=(nn,),
        in_specs=[
            pl.BlockSpec(z.shape, const(0, 0)),
            pl.BlockSpec(z1.shape, const(0, 0)),
            pl.BlockSpec(cb_x.shape, const(0, 0)),
            pl.BlockSpec(cb_y.shape, const(0, 0)),
            pl.BlockSpec(W1.shape, const(0, 0)),
            pl.BlockSpec((1, W1.shape[0]), const(0, 0)),
            pl.BlockSpec((1, W1.shape[0]), const(0, 0)),
            pl.BlockSpec((1, W1.shape[0]), const(0, 0)),
            pl.BlockSpec(W2.shape, const(0, 0)),
            pl.BlockSpec((1, W2.shape[0]), const(0, 0)),
            pl.BlockSpec((1, W2.shape[0]), const(0, 0)),
            pl.BlockSpec((1, W2.shape[0]), const(0, 0)),
            pl.BlockSpec(W3.shape, const(0, 0)),
            pl.BlockSpec((1, W3.shape[0]), const(0, 0)),
            pl.BlockSpec((1, W3.shape[0]), const(0, 0)),
            pl.BlockSpec((1, W3.shape[0]), const(0, 0)),
            pl.BlockSpec((n_blk, W4.shape[1]), lambda n: (n, 0)),
            pl.BlockSpec((1, n_blk), lambda n: (0, n)),
            pl.BlockSpec((B, n_blk), lambda n: (0, n)),
            pl.BlockSpec((B, n_blk), lambda n: (0, n)),
        ],
        out_specs=pl.BlockSpec((1, 1), lambda n: (0, 0)),
        out_shape=jax.ShapeDtypeStruct((1, 1), jnp.float32),
        scratch_shapes=[pltpu.VMEM((B2, W3.shape[0]), jnp.bfloat16)],
    )(z, z1, cb_x, cb_y,
      W1, row(p["b"][0]), row(p["g"][0]), row(p["beta"][0]),
      W2, row(p["b"][1]), row(p["g"][1]), row(p["beta"][1]),
      W3, row(p["b"][2]), row(p["g"][2]), row(p["beta"][2]),
      W4, row(p["b"][3]), t_y, t_x)


def kernel(x, y, params):
    z = _encoder(x, params["enc_x"])
    z1 = _encoder(y, params["enc_y"])
    total = _decoder_loss(z, z1, params["cb_x"], params["cb_y"],
                          params["dec"], y, x)
    return total[0, 0]
